# Initial kernel scaffold; baseline (speedup 1.0000x reference)
import jax
import jax.numpy as jnp
from jax.experimental import pallas as pl


def kernel(h, edge_index, W1, b1, W2, b2):
    raise NotImplementedError("write your pallas kernel here")


# speedup vs baseline: 4.8371x; 4.8371x over previous
import functools

import jax
import jax.numpy as jnp
from jax import lax
from jax.experimental import pallas as pl
from jax.experimental.pallas import tpu as pltpu
from jax.experimental.pallas import tpu_sc as plsc

N_NODES = 10000
NP = 10240
CHUNK = 128
NCH = 79
CPT = CHUNK * NCH
NW = 32
EPAD = CPT * NW
RPT = NP // 16


def _sc_agg(D):
  mesh = plsc.VectorSubcoreMesh(core_axis_name="c", subcore_axis_name="s")

  @functools.partial(
      pl.kernel,
      out_type=jax.ShapeDtypeStruct((2, NP, D), jnp.float32),
      mesh=mesh,
      scratch_types=[
          pltpu.VMEM((NCH, CHUNK), jnp.int32),
          pltpu.VMEM((NCH, CHUNK), jnp.int32),
          pltpu.VMEM((CHUNK, D), jnp.float32),
          pltpu.VMEM_SHARED((NP, D), jnp.float32),
          pltpu.SemaphoreType.DMA,
      ],
      compiler_params=pltpu.CompilerParams(use_tc_tiling_on_sc=False),
  )
  def k(h_hbm, src_hbm, dst_hbm, zero_hbm, out_hbm, src_v, dst_v, rows_v,
        acc_s, sem):
    cid = lax.axis_index("c")
    sid = lax.axis_index("s")
    wid = cid * 16 + sid

    pltpu.sync_copy(src_hbm.at[wid], src_v)
    pltpu.sync_copy(dst_hbm.at[wid], dst_v)

    pltpu.sync_copy(zero_hbm, rows_v)
    for j in range(RPT // CHUNK):
      pltpu.sync_copy(rows_v, acc_s.at[pl.ds(sid * RPT + j * CHUNK, CHUNK)])
    plsc.subcore_barrier()

    def body(c, carry):
      pltpu.async_copy(h_hbm.at[src_v.at[c]], rows_v, sem).wait()
      pltpu.sync_copy(rows_v, acc_s.at[dst_v.at[c]], add=True)
      return carry

    lax.fori_loop(0, NCH, body, 0)
    plsc.subcore_barrier()

    def wbody(j, carry):
      r = sid * RPT + j * CHUNK
      pltpu.sync_copy(acc_s.at[pl.ds(r, CHUNK)], rows_v)
      pltpu.sync_copy(rows_v, out_hbm.at[cid, pl.ds(r, CHUNK)])
      return carry

    lax.fori_loop(0, RPT // CHUNK, wbody, 0)

  return k


_R = 1024


def _tc1(p, w, b):
  def body(p_ref, w_ref, b_ref, h_ref, rc_ref):
    s = p_ref[0] + p_ref[1]
    rc = 1.0 / jnp.maximum(s[:, 128:129], 1.0)
    x = s[:, :128] * rc
    y = lax.dot_general(x, w_ref[...], (((1,), (1,)), ((), ())),
                        preferred_element_type=jnp.float32) + b_ref[...]
    h_ref[...] = jnp.where(y > 0, y, jnp.exp(jnp.minimum(y, 0.0)) - 1.0)
    rc_ref[...] = rc

  return pl.pallas_call(
      body,
      grid=(NP // _R,),
      in_specs=[
          pl.BlockSpec((2, _R, 144), lambda i: (0, i, 0)),
          pl.BlockSpec((128, 128), lambda i: (0, 0)),
          pl.BlockSpec((1, 128), lambda i: (0, 0)),
      ],
      out_specs=[
          pl.BlockSpec((_R, 128), lambda i: (i, 0)),
          pl.BlockSpec((_R, 1), lambda i: (i, 0)),
      ],
      out_shape=[
          jax.ShapeDtypeStruct((NP, 128), jnp.float32),
          jax.ShapeDtypeStruct((NP, 1), jnp.float32),
      ],
  )(p, w, b)


def _tc2(p, rc, w, b):
  def body(p_ref, rc_ref, w_ref, b_ref, o_ref):
    x = (p_ref[0] + p_ref[1]) * rc_ref[...]
    y = lax.dot_general(x, w_ref[...], (((1,), (1,)), ((), ())),
                        preferred_element_type=jnp.float32) + b_ref[...]
    o_ref[...] = jnp.where(y > 0, y, jnp.exp(jnp.minimum(y, 0.0)) - 1.0)

  return pl.pallas_call(
      body,
      grid=(NP // _R,),
      in_specs=[
          pl.BlockSpec((2, _R, 128), lambda i: (0, i, 0)),
          pl.BlockSpec((_R, 1), lambda i: (i, 0)),
          pl.BlockSpec((128, 128), lambda i: (0, 0)),
          pl.BlockSpec((1, 128), lambda i: (0, 0)),
      ],
      out_specs=pl.BlockSpec((_R, 128), lambda i: (i, 0)),
      out_shape=jax.ShapeDtypeStruct((NP, 128), jnp.float32),
  )(p, rc, w, b)


def kernel(h, edge_index, W1, b1, W2, b2):
  E = edge_index.shape[1]
  pad = EPAD - E
  src = jnp.concatenate([edge_index[0].astype(jnp.int32),
                         jnp.zeros((pad,), jnp.int32)]).reshape(NW, NCH, CHUNK)
  dst = jnp.concatenate([edge_index[1].astype(jnp.int32),
                         jnp.full((pad,), N_NODES, jnp.int32)]
                        ).reshape(NW, NCH, CHUNK)
  h_aug = jnp.concatenate(
      [h, jnp.ones((N_NODES, 1), jnp.float32),
       jnp.zeros((N_NODES, 15), jnp.float32)], axis=1)

  z144 = jnp.zeros((CHUNK, 144), jnp.float32)
  z128 = jnp.zeros((CHUNK, 128), jnp.float32)

  p1 = _sc_agg(144)(h_aug, src, dst, z144)
  h1, rc = _tc1(p1, W1, b1.reshape(1, 128))
  p2 = _sc_agg(128)(h1, src, dst, z128)
  out = _tc2(p2, rc, W2, b2.reshape(1, 128))
  return out[:N_NODES]


# speedup vs baseline: 4.9750x; 1.0285x over previous
import functools

import jax
import jax.numpy as jnp
from jax import lax
from jax.experimental import pallas as pl
from jax.experimental.pallas import tpu as pltpu
from jax.experimental.pallas import tpu_sc as plsc

N_NODES = 10000
NP = 10112
CHUNK = 64
NCH = 158
CPT = CHUNK * NCH
NW = 32
EPAD = CPT * NW
RPT = NP // 16


def _sc_agg(D):
  mesh = plsc.VectorSubcoreMesh(core_axis_name="c", subcore_axis_name="s")

  @functools.partial(
      pl.kernel,
      out_type=jax.ShapeDtypeStruct((2, NP, D), jnp.float32),
      mesh=mesh,
      scratch_types=[
          pltpu.VMEM((NCH, CHUNK), jnp.int32),
          pltpu.VMEM((NCH, CHUNK), jnp.int32),
          pltpu.VMEM((2, CHUNK, D), jnp.float32),
          pltpu.VMEM_SHARED((NP, D), jnp.float32),
          pltpu.SemaphoreType.DMA,
          pltpu.SemaphoreType.DMA,
      ],
      compiler_params=pltpu.CompilerParams(use_tc_tiling_on_sc=False),
  )
  def k(h_hbm, src_hbm, dst_hbm, zero_hbm, out_hbm, src_v, dst_v, rows_v,
        acc_s, gsem, ssem):
    cid = lax.axis_index("c")
    sid = lax.axis_index("s")
    wid = cid * 16 + sid

    pltpu.sync_copy(src_hbm.at[wid], src_v)
    pltpu.sync_copy(dst_hbm.at[wid], dst_v)

    pltpu.sync_copy(zero_hbm, rows_v.at[0])
    def zbody(j, carry):
      pltpu.sync_copy(rows_v.at[0], acc_s.at[pl.ds(sid * RPT + j * CHUNK, CHUNK)])
      return carry
    lax.fori_loop(0, RPT // CHUNK, zbody, 0)
    pltpu.sync_copy(rows_v.at[0, pl.ds(0, RPT - (RPT // CHUNK) * CHUNK)],
                    acc_s.at[pl.ds(sid * RPT + (RPT // CHUNK) * CHUNK,
                                   RPT - (RPT // CHUNK) * CHUNK)])
    plsc.subcore_barrier()

    pltpu.async_copy(h_hbm.at[src_v.at[0]], rows_v.at[0], gsem)

    def body(c, carry):
      b = lax.rem(c, 2)
      nb = 1 - b
      pltpu.make_async_copy(h_hbm.at[src_v.at[c]], rows_v.at[b], gsem).wait()
      @pl.when(c + 1 < NCH)
      def _():
        pltpu.async_copy(h_hbm.at[src_v.at[c + 1]], rows_v.at[nb], gsem)
      pltpu.sync_copy(rows_v.at[b], acc_s.at[dst_v.at[c]], add=True)
      return carry

    lax.fori_loop(0, NCH, body, 0)
    plsc.subcore_barrier()

    def wbody(j, carry):
      r = sid * RPT + j * CHUNK
      pltpu.sync_copy(acc_s.at[pl.ds(r, CHUNK)], rows_v.at[0])
      pltpu.sync_copy(rows_v.at[0], out_hbm.at[cid, pl.ds(r, CHUNK)])
      return carry

    lax.fori_loop(0, RPT // CHUNK, wbody, 0)
    _tail = RPT - (RPT // CHUNK) * CHUNK
    r = sid * RPT + (RPT // CHUNK) * CHUNK
    pltpu.sync_copy(acc_s.at[pl.ds(r, _tail)], rows_v.at[0, pl.ds(0, _tail)])
    pltpu.sync_copy(rows_v.at[0, pl.ds(0, _tail)],
                    out_hbm.at[cid, pl.ds(r, _tail)])

  return k


_R = 1264


def _tc1(p, w, b):
  def body(p_ref, w_ref, b_ref, h_ref, rc_ref):
    s = p_ref[0] + p_ref[1]
    rc = 1.0 / jnp.maximum(s[:, 128:129], 1.0)
    x = s[:, :128] * rc
    y = lax.dot_general(x, w_ref[...], (((1,), (1,)), ((), ())),
                        preferred_element_type=jnp.float32) + b_ref[...]
    h_ref[...] = jnp.where(y > 0, y, jnp.exp(jnp.minimum(y, 0.0)) - 1.0)
    rc_ref[...] = rc

  return pl.pallas_call(
      body,
      grid=(NP // _R,),
      in_specs=[
          pl.BlockSpec((2, _R, 144), lambda i: (0, i, 0)),
          pl.BlockSpec((128, 128), lambda i: (0, 0)),
          pl.BlockSpec((1, 128), lambda i: (0, 0)),
      ],
      out_specs=[
          pl.BlockSpec((_R, 128), lambda i: (i, 0)),
          pl.BlockSpec((_R, 1), lambda i: (i, 0)),
      ],
      out_shape=[
          jax.ShapeDtypeStruct((NP, 128), jnp.float32),
          jax.ShapeDtypeStruct((NP, 1), jnp.float32),
      ],
  )(p, w, b)


def _tc2(p, rc, w, b):
  def body(p_ref, rc_ref, w_ref, b_ref, o_ref):
    x = (p_ref[0] + p_ref[1]) * rc_ref[...]
    y = lax.dot_general(x, w_ref[...], (((1,), (1,)), ((), ())),
                        preferred_element_type=jnp.float32) + b_ref[...]
    o_ref[...] = jnp.where(y > 0, y, jnp.exp(jnp.minimum(y, 0.0)) - 1.0)

  return pl.pallas_call(
      body,
      grid=(NP // _R,),
      in_specs=[
          pl.BlockSpec((2, _R, 128), lambda i: (0, i, 0)),
          pl.BlockSpec((_R, 1), lambda i: (i, 0)),
          pl.BlockSpec((128, 128), lambda i: (0, 0)),
          pl.BlockSpec((1, 128), lambda i: (0, 0)),
      ],
      out_specs=pl.BlockSpec((_R, 128), lambda i: (i, 0)),
      out_shape=jax.ShapeDtypeStruct((NP, 128), jnp.float32),
  )(p, rc, w, b)


def kernel(h, edge_index, W1, b1, W2, b2):
  E = edge_index.shape[1]
  pad = EPAD - E
  src = jnp.concatenate([edge_index[0].astype(jnp.int32),
                         jnp.zeros((pad,), jnp.int32)]).reshape(NW, NCH, CHUNK)
  dst = jnp.concatenate([edge_index[1].astype(jnp.int32),
                         jnp.full((pad,), N_NODES, jnp.int32)]
                        ).reshape(NW, NCH, CHUNK)
  h_aug = jnp.concatenate(
      [h, jnp.ones((N_NODES, 1), jnp.float32),
       jnp.zeros((N_NODES, 15), jnp.float32)], axis=1)

  z144 = jnp.zeros((CHUNK, 144), jnp.float32)
  z128 = jnp.zeros((CHUNK, 128), jnp.float32)

  p1 = _sc_agg(144)(h_aug, src, dst, z144)
  h1, rc = _tc1(p1, W1, b1.reshape(1, 128))
  p2 = _sc_agg(128)(h1, src, dst, z128)
  out = _tc2(p2, rc, W2, b2.reshape(1, 128))
  return out[:N_NODES]
